# baseline (speedup 1.0000x reference)
import functools

import jax
import jax.numpy as jnp
from jax import lax
from jax.experimental import pallas as pl
from jax.experimental.pallas import tpu as pltpu
from jax.experimental.pallas import tpu_sc as plsc

N_ROWS = 16384
N_BITS = 12
LUT_SIZE = 4096
WORDS_PER_ROW = LUT_SIZE // 4


@functools.cache
def _build_call():
    info = plsc.get_sparse_core_info()
    nc, ns, lanes = info.num_cores, info.num_subcores, info.num_lanes
    nw = nc * ns
    rows_w = N_ROWS // nw
    chunks = rows_w // lanes
    idx_rows = rows_w // 128
    mesh = plsc.VectorSubcoreMesh(core_axis_name="c", subcore_axis_name="s")

    @functools.partial(
        pl.kernel,
        mesh=mesh,
        out_type=jax.ShapeDtypeStruct((N_ROWS,), jnp.float32),
        scratch_types=[
            pltpu.VMEM((N_BITS, rows_w), jnp.int32),
            pltpu.VMEM((idx_rows, 128), jnp.int32),
            pltpu.VMEM((rows_w,), jnp.int32),
            pltpu.VMEM((idx_rows, 128), jnp.int32),
            pltpu.VMEM((rows_w,), jnp.float32),
            pltpu.SemaphoreType.DMA,
        ],
    )
    def lut_fwd(bits_hbm, table_hbm, out_hbm, bits_v, idx_v, bsh_v, vals_v,
                out_v, sem):
        wid = lax.axis_index("s") * nc + lax.axis_index("c")
        base = wid * rows_w
        pltpu.sync_copy(bits_hbm.at[:, pl.ds(base, rows_w)], bits_v)

        for c in range(chunks):
            addr = bits_v[0, pl.ds(c * lanes, lanes)]
            for j in range(1, N_BITS):
                addr = addr + (bits_v[j, pl.ds(c * lanes, lanes)] << j)
            row = base + c * lanes + lax.iota(jnp.int32, lanes)
            idx_v[c // 8, pl.ds((c % 8) * lanes, lanes)] = (
                row * WORDS_PER_ROW + (addr >> 2)
            )
            bsh_v[pl.ds(c * lanes, lanes)] = (addr & 3) << 3

        copies = [
            pltpu.async_copy(table_hbm.at[idx_v.at[t]], vals_v.at[t], sem)
            for t in range(idx_rows)
        ]
        for cp in copies:
            cp.wait()

        for c in range(chunks):
            word = vals_v[c // 8, pl.ds((c % 8) * lanes, lanes)]
            b = (word >> bsh_v[pl.ds(c * lanes, lanes)]) & 255
            b = b - ((b >> 7) << 8)
            out_v[pl.ds(c * lanes, lanes)] = b.astype(jnp.float32)

        pltpu.sync_copy(out_v, out_hbm.at[pl.ds(base, rows_w)])

    return lut_fwd


def kernel(inputs, luts_float, luts_int):
    del luts_float
    bits = jnp.transpose(jnp.reshape(inputs, (N_ROWS, N_BITS))).astype(jnp.int32)
    flat = jnp.reshape(luts_int, (-1,))
    b0 = flat[0::4].astype(jnp.int32) & 255
    b1 = flat[1::4].astype(jnp.int32) & 255
    b2 = flat[2::4].astype(jnp.int32) & 255
    b3 = flat[3::4].astype(jnp.int32) & 255
    table = b0 | (b1 << 8) | (b2 << 16) | (b3 << 24)
    y = _build_call()(bits, table)
    return jnp.reshape(y, (N_ROWS, 1))


# speedup vs baseline: 36.8423x; 36.8423x over previous
import functools

import jax
import jax.numpy as jnp
from jax import lax
from jax.experimental import pallas as pl
from jax.experimental.pallas import tpu as pltpu
from jax.experimental.pallas import tpu_sc as plsc

N_ROWS = 16384
N_BITS = 12
LUT_SIZE = 4096
WORDS_PER_ROW = LUT_SIZE // 4


@functools.cache
def _build_call():
    info = plsc.get_sparse_core_info()
    nc, ns, lanes = info.num_cores, info.num_subcores, info.num_lanes
    nw = nc * ns
    rows_w = N_ROWS // nw
    chunks = rows_w // lanes
    idx_rows = rows_w // 128
    mesh = plsc.VectorSubcoreMesh(core_axis_name="c", subcore_axis_name="s")

    @functools.partial(
        pl.kernel,
        mesh=mesh,
        out_type=jax.ShapeDtypeStruct((N_ROWS,), jnp.float32),
        scratch_types=[
            pltpu.VMEM((N_BITS, rows_w), jnp.int32),
            pltpu.VMEM((idx_rows, 128), jnp.int32),
            pltpu.VMEM((idx_rows, 128), jnp.int32),
            pltpu.VMEM((rows_w,), jnp.float32),
            pltpu.SemaphoreType.DMA,
        ],
    )
    def lut_fwd(bits_hbm, table_hbm, out_hbm, bits_v, idx_v, vals_v,
                out_v, sem):
        wid = lax.axis_index("s") * nc + lax.axis_index("c")
        base = wid * rows_w
        pltpu.sync_copy(bits_hbm.at[:, pl.ds(base, rows_w)], bits_v)

        for c in range(chunks):
            addr = bits_v[0, pl.ds(c * lanes, lanes)]
            for j in range(1, N_BITS):
                addr = addr + (bits_v[j, pl.ds(c * lanes, lanes)] << j)
            row = base + c * lanes + lax.iota(jnp.int32, lanes)
            idx_v[c // 8, pl.ds((c % 8) * lanes, lanes)] = (
                row * LUT_SIZE + addr
            )

        copies = [
            pltpu.async_copy(table_hbm.at[idx_v.at[t]], vals_v.at[t], sem)
            for t in range(idx_rows)
        ]
        for cp in copies:
            cp.wait()

        for c in range(chunks):
            val = vals_v[c // 8, pl.ds((c % 8) * lanes, lanes)]
            out_v[pl.ds(c * lanes, lanes)] = val.astype(jnp.float32)

        pltpu.sync_copy(out_v, out_hbm.at[pl.ds(base, rows_w)])

    return lut_fwd


def kernel(inputs, luts_float, luts_int):
    del luts_float
    bits = jnp.transpose(jnp.reshape(inputs, (N_ROWS, N_BITS))).astype(jnp.int32)
    table = jnp.reshape(luts_int.astype(jnp.int32), (-1,))
    y = _build_call()(bits, table)
    return jnp.reshape(y, (N_ROWS, 1))


# speedup vs baseline: 56.6001x; 1.5363x over previous
import functools

import jax
import jax.numpy as jnp
from jax import lax
from jax.experimental import pallas as pl
from jax.experimental.pallas import tpu as pltpu
from jax.experimental.pallas import tpu_sc as plsc

N_ROWS = 16384
N_BITS = 12
LUT_SIZE = 4096

_MAGIC = 12582912.0


@functools.cache
def _build_call():
    info = plsc.get_sparse_core_info()
    nc, ns, lanes = info.num_cores, info.num_subcores, info.num_lanes
    nw = nc * ns
    rows_w = N_ROWS // nw
    chunks = rows_w // lanes
    idx_rows = rows_w // 128
    mesh = plsc.VectorSubcoreMesh(core_axis_name="c", subcore_axis_name="s")

    @functools.partial(
        pl.kernel,
        mesh=mesh,
        out_type=jax.ShapeDtypeStruct((N_ROWS,), jnp.float32),
        scratch_types=[
            pltpu.VMEM((N_BITS, rows_w), jnp.int32),
            pltpu.VMEM((idx_rows, 128), jnp.int32),
            pltpu.VMEM((idx_rows, 128), jnp.float32),
            pltpu.VMEM((rows_w,), jnp.float32),
            pltpu.SemaphoreType.DMA,
        ],
    )
    def lut_fwd(bits_hbm, table_hbm, out_hbm, bits_v, idx_v, vals_v,
                out_v, sem):
        wid = lax.axis_index("s") * nc + lax.axis_index("c")
        base = wid * rows_w
        pltpu.sync_copy(bits_hbm.at[:, pl.ds(base, rows_w)], bits_v)

        for c in range(chunks):
            addr = bits_v[0, pl.ds(c * lanes, lanes)]
            for j in range(1, N_BITS):
                addr = addr + (bits_v[j, pl.ds(c * lanes, lanes)] << j)
            row = base + c * lanes + lax.iota(jnp.int32, lanes)
            idx_v[c // 8, pl.ds((c % 8) * lanes, lanes)] = (
                row * LUT_SIZE + addr
            )

        copies = [
            pltpu.async_copy(table_hbm.at[idx_v.at[t]], vals_v.at[t], sem)
            for t in range(idx_rows)
        ]
        for cp in copies:
            cp.wait()

        for c in range(chunks):
            v = vals_v[c // 8, pl.ds((c % 8) * lanes, lanes)]
            out_v[pl.ds(c * lanes, lanes)] = ((v + 0.5) + _MAGIC) - _MAGIC

        pltpu.sync_copy(out_v, out_hbm.at[pl.ds(base, rows_w)])

    return lut_fwd


def kernel(inputs, luts_float, luts_int):
    del luts_int
    bits = jnp.transpose(jnp.reshape(inputs, (N_ROWS, N_BITS))).astype(jnp.int32)
    table = jnp.reshape(luts_float, (-1,))
    y = _build_call()(bits, table)
    return jnp.reshape(y, (N_ROWS, 1))
